# baseline (speedup 1.0000x reference)
import functools

import jax
import jax.numpy as jnp
from jax import lax
from jax.experimental import pallas as pl
from jax.experimental.pallas import tpu as pltpu
from jax.experimental.pallas import tpu_sc as plsc

N = 10000
E = 320000
D = 128
H = 128
C = 40
C_PAD = 48

N_PAD = 10240
NC = 2
NS = 16
NW = NC * NS
EPW = E // NW
CHUNK = 96
CHUNKS = 105
EPW_PAD = CHUNKS * CHUNK
ROWS_PER_TILE = N_PAD // NS

_mesh = plsc.VectorSubcoreMesh(
    core_axis_name="c", subcore_axis_name="s", num_cores=NC, num_subcores=NS)


@functools.partial(
    pl.kernel,
    out_type=jax.ShapeDtypeStruct((NW, N_PAD), jnp.float32),
    mesh=_mesh,
    scratch_types=[
        pltpu.VMEM((EPW_PAD,), jnp.int32),
        pltpu.VMEM((N_PAD,), jnp.float32),
    ],
    compiler_params=pltpu.CompilerParams(
        needs_layout_passes=False, use_tc_tiling_on_sc=False),
)
def _deg_kernel(dst_hbm, deg_out, dst_v, deg_v):
    c = lax.axis_index("c")
    s = lax.axis_index("s")
    wid = s * NC + c
    pltpu.sync_copy(dst_hbm.at[wid], dst_v)

    zeros16 = jnp.zeros((16,), jnp.float32)

    def zero_body(i, _):
        deg_v[pl.ds(i * 16, 16)] = zeros16
        return 0

    lax.fori_loop(0, N_PAD // 16, zero_body, 0)

    ones16 = jnp.ones((16,), jnp.float32)

    def body(i, _):
        idx = dst_v[pl.ds(i * 16, 16)]
        plsc.addupdate_scatter(deg_v, [idx], ones16)
        return 0

    lax.fori_loop(0, EPW_PAD // 16, body, 0)

    pltpu.sync_copy(deg_v, deg_out.at[wid])


def _make_prop(width):
    @functools.partial(
        pl.kernel,
        out_type=jax.ShapeDtypeStruct((NC, N_PAD, width), jnp.float32),
        mesh=_mesh,
        scratch_types=[
            pltpu.VMEM((CHUNKS, CHUNK), jnp.int32),
            pltpu.VMEM((CHUNKS, CHUNK), jnp.int32),
            pltpu.VMEM((CHUNK, width), jnp.float32),
            pltpu.VMEM((CHUNK, width), jnp.float32),
            pltpu.VMEM_SHARED((N_PAD, width), jnp.float32),
            pltpu.SemaphoreType.DMA,
            pltpu.SemaphoreType.DMA,
        ],
        compiler_params=pltpu.CompilerParams(use_tc_tiling_on_sc=False),
    )
    def _prop(u_hbm, src_hbm, dst_hbm, out_hbm, src_v, dst_v, rows_a, rows_b,
              acc_sh, sem_a, sem_b):
        c = lax.axis_index("c")
        s = lax.axis_index("s")
        wid = s * NC + c
        pltpu.sync_copy(src_hbm.at[wid], src_v)
        pltpu.sync_copy(dst_hbm.at[wid], dst_v)

        zeros16 = jnp.zeros((16,), jnp.float32)
        lanes = width // 16

        def zero_body(i, _):
            for k in range(lanes):
                rows_a[i, pl.ds(k * 16, 16)] = zeros16
            return 0

        lax.fori_loop(0, CHUNK, zero_body, 0)

        for k in range(ROWS_PER_TILE // CHUNK):
            pltpu.sync_copy(
                rows_a, acc_sh.at[pl.ds(s * ROWS_PER_TILE + k * CHUNK, CHUNK)])
        plsc.subcore_barrier()

        def body(j, _):
            pltpu.async_copy(u_hbm.at[src_v.at[j]], rows_a, sem_a).wait()
            pltpu.sync_copy(rows_a, acc_sh.at[dst_v.at[j]], add=True)
            return 0

        lax.fori_loop(0, CHUNKS, body, 0)

        plsc.subcore_barrier()
        pltpu.sync_copy(
            acc_sh.at[pl.ds(s * ROWS_PER_TILE, ROWS_PER_TILE)],
            out_hbm.at[c, pl.ds(s * ROWS_PER_TILE, ROWS_PER_TILE)])

    return _prop


_prop128 = _make_prop(H)
_prop64 = _make_prop(C_PAD)


_BR = 512
_GRID = N_PAD // _BR


def _pre1_body(x_ref, w_ref, degt_ref, u_ref, dinv_ref):
    deg = jnp.sum(degt_ref[...], axis=1, keepdims=True) + 1.0
    dinv = lax.rsqrt(deg)
    z = jnp.dot(x_ref[...], w_ref[...], preferred_element_type=jnp.float32)
    u_ref[...] = dinv * z
    dinv_ref[...] = dinv


def _tc_pre1(x_pad, W1, degT):
    return pl.pallas_call(
        _pre1_body,
        grid=(_GRID,),
        in_specs=[
            pl.BlockSpec((_BR, D), lambda i: (i, 0)),
            pl.BlockSpec((D, H), lambda i: (0, 0)),
            pl.BlockSpec((_BR, NW), lambda i: (i, 0)),
        ],
        out_specs=[
            pl.BlockSpec((_BR, H), lambda i: (i, 0)),
            pl.BlockSpec((_BR, 1), lambda i: (i, 0)),
        ],
        out_shape=[
            jax.ShapeDtypeStruct((N_PAD, H), jnp.float32),
            jax.ShapeDtypeStruct((N_PAD, 1), jnp.float32),
        ],
    )(x_pad, W1, degT)


def _mid_body(s0_ref, s1_ref, u_ref, dinv_ref, b_ref, w_ref, out_ref):
    dinv = dinv_ref[...]
    z = dinv * (s0_ref[...] + s1_ref[...] + u_ref[...]) + b_ref[...]
    h = jnp.maximum(z, 0.0)
    out_ref[...] = dinv * jnp.dot(h, w_ref[...],
                                  preferred_element_type=jnp.float32)


def _tc_mid(s0, s1, u, dinv, b, w, w_out):
    win = u.shape[1]
    return pl.pallas_call(
        _mid_body,
        grid=(_GRID,),
        in_specs=[
            pl.BlockSpec((_BR, win), lambda i: (i, 0)),
            pl.BlockSpec((_BR, win), lambda i: (i, 0)),
            pl.BlockSpec((_BR, win), lambda i: (i, 0)),
            pl.BlockSpec((_BR, 1), lambda i: (i, 0)),
            pl.BlockSpec((win,), lambda i: (0,)),
            pl.BlockSpec((win, w_out), lambda i: (0, 0)),
        ],
        out_specs=pl.BlockSpec((_BR, w_out), lambda i: (i, 0)),
        out_shape=jax.ShapeDtypeStruct((N_PAD, w_out), jnp.float32),
    )(s0, s1, u, dinv, b, w)


def _post_body(s0_ref, s1_ref, u_ref, dinv_ref, b_ref, out_ref):
    z = dinv_ref[...] * (s0_ref[...] + s1_ref[...] + u_ref[...]) + b_ref[...]
    m = jnp.max(z, axis=1, keepdims=True)
    e = jnp.exp(z - m)
    ssum = jnp.sum(e, axis=1, keepdims=True)
    out_ref[...] = z - m - jnp.log(ssum)


def _tc_post(s0, s1, u, dinv, b3p):
    return pl.pallas_call(
        _post_body,
        grid=(_GRID,),
        in_specs=[
            pl.BlockSpec((_BR, C_PAD), lambda i: (i, 0)),
            pl.BlockSpec((_BR, C_PAD), lambda i: (i, 0)),
            pl.BlockSpec((_BR, C_PAD), lambda i: (i, 0)),
            pl.BlockSpec((_BR, 1), lambda i: (i, 0)),
            pl.BlockSpec((C_PAD,), lambda i: (0,)),
        ],
        out_specs=pl.BlockSpec((_BR, C_PAD), lambda i: (i, 0)),
        out_shape=jax.ShapeDtypeStruct((N_PAD, C_PAD), jnp.float32),
    )(s0, s1, u, dinv, b3p)


def kernel(x, edge_index, W1, b1, W2, b2, W3, b3):
    pad_e = jnp.full((NW, EPW_PAD - EPW), N_PAD - 1, dtype=jnp.int32)
    src = jnp.concatenate(
        [edge_index[0].reshape(NW, EPW), pad_e], axis=1)
    dst = jnp.concatenate(
        [edge_index[1].reshape(NW, EPW), pad_e], axis=1)
    dst_flat = dst
    src = src.reshape(NW, CHUNKS, CHUNK)
    dst = dst.reshape(NW, CHUNKS, CHUNK)
    x_pad = jnp.pad(x, ((0, N_PAD - N), (0, 0)))
    W3p = jnp.pad(W3, ((0, 0), (0, C_PAD - C)))
    b3p = jnp.pad(b3, (0, C_PAD - C), constant_values=-1e30)

    degp = _deg_kernel(dst_flat)
    degT = degp.T

    u1, dinv = _tc_pre1(x_pad, W1, degT)
    s1 = _prop128(u1, src, dst)
    u2 = _tc_mid(s1[0], s1[1], u1, dinv, b1, W2, H)
    s2 = _prop128(u2, src, dst)
    u3 = _tc_mid(s2[0], s2[1], u2, dinv, b2, W3p, C_PAD)
    s3 = _prop64(u3, src, dst)
    out = _tc_post(s3[0], s3[1], u3, dinv, b3p)
    return out[:N, :C]


# speedup vs baseline: 1.8439x; 1.8439x over previous
import functools

import jax
import jax.numpy as jnp
from jax import lax
from jax.experimental import pallas as pl
from jax.experimental.pallas import tpu as pltpu
from jax.experimental.pallas import tpu_sc as plsc

N = 10000
E = 320000
D = 128
H = 128
C = 40
C_PAD = 64

N_PAD = 10240
NC = 2
NS = 16
NW = NC * NS
EPW = E // NW
CHUNK = 80
CHUNKS = 125
EPW_PAD = CHUNKS * CHUNK
ROWS_PER_TILE = N_PAD // NS

_mesh = plsc.VectorSubcoreMesh(
    core_axis_name="c", subcore_axis_name="s", num_cores=NC, num_subcores=NS)


@functools.partial(
    pl.kernel,
    out_type=jax.ShapeDtypeStruct((NW, N_PAD), jnp.float32),
    mesh=_mesh,
    scratch_types=[
        pltpu.VMEM((EPW_PAD,), jnp.int32),
        pltpu.VMEM((N_PAD,), jnp.float32),
    ],
    compiler_params=pltpu.CompilerParams(
        needs_layout_passes=False, use_tc_tiling_on_sc=False),
)
def _deg_kernel(dst_hbm, deg_out, dst_v, deg_v):
    c = lax.axis_index("c")
    s = lax.axis_index("s")
    wid = s * NC + c
    pltpu.sync_copy(dst_hbm.at[wid], dst_v)

    zeros16 = jnp.zeros((16,), jnp.float32)

    def zero_body(i, _):
        deg_v[pl.ds(i * 16, 16)] = zeros16
        return 0

    lax.fori_loop(0, N_PAD // 16, zero_body, 0)

    ones16 = jnp.ones((16,), jnp.float32)

    def body(i, _):
        idx = dst_v[pl.ds(i * 16, 16)]
        plsc.addupdate_scatter(deg_v, [idx], ones16)
        return 0

    lax.fori_loop(0, EPW_PAD // 16, body, 0)

    pltpu.sync_copy(deg_v, deg_out.at[wid])


def _make_prop(width):
    @functools.partial(
        pl.kernel,
        out_type=jax.ShapeDtypeStruct((NC, N_PAD, width), jnp.float32),
        mesh=_mesh,
        scratch_types=[
            pltpu.VMEM((CHUNKS, CHUNK), jnp.int32),
            pltpu.VMEM((CHUNKS, CHUNK), jnp.int32),
            pltpu.VMEM((CHUNK, width), jnp.float32),
            pltpu.VMEM((CHUNK, width), jnp.float32),
            pltpu.VMEM_SHARED((N_PAD, width), jnp.float32),
            pltpu.SemaphoreType.DMA,
            pltpu.SemaphoreType.DMA,
        ],
        compiler_params=pltpu.CompilerParams(use_tc_tiling_on_sc=False),
    )
    def _prop(u_hbm, src_hbm, dst_hbm, out_hbm, src_v, dst_v, rows_a, rows_b,
              acc_sh, sem_a, sem_b):
        c = lax.axis_index("c")
        s = lax.axis_index("s")
        wid = s * NC + c
        pltpu.sync_copy(src_hbm.at[wid], src_v)
        pltpu.sync_copy(dst_hbm.at[wid], dst_v)

        zeros16 = jnp.zeros((16,), jnp.float32)
        lanes = width // 16

        def zero_body(i, _):
            for k in range(lanes):
                rows_a[i, pl.ds(k * 16, 16)] = zeros16
            return 0

        lax.fori_loop(0, CHUNK, zero_body, 0)

        for k in range(ROWS_PER_TILE // CHUNK):
            pltpu.sync_copy(
                rows_a, acc_sh.at[pl.ds(s * ROWS_PER_TILE + k * CHUNK, CHUNK)])
        plsc.subcore_barrier()

        pltpu.async_copy(u_hbm.at[src_v.at[0]], rows_a, sem_a)

        def body(kk, _):
            j = 2 * kk
            pltpu.async_copy(u_hbm.at[src_v.at[j + 1]], rows_b, sem_b)
            pltpu.make_async_copy(u_hbm.at[src_v.at[j]], rows_a, sem_a).wait()
            pltpu.sync_copy(rows_a, acc_sh.at[dst_v.at[j]], add=True)
            pltpu.async_copy(u_hbm.at[src_v.at[j + 2]], rows_a, sem_a)
            pltpu.make_async_copy(
                u_hbm.at[src_v.at[j + 1]], rows_b, sem_b).wait()
            pltpu.sync_copy(rows_b, acc_sh.at[dst_v.at[j + 1]], add=True)
            return 0

        lax.fori_loop(0, (CHUNKS - 1) // 2, body, 0)
        pltpu.make_async_copy(
            u_hbm.at[src_v.at[CHUNKS - 1]], rows_a, sem_a).wait()
        pltpu.sync_copy(rows_a, acc_sh.at[dst_v.at[CHUNKS - 1]], add=True)

        plsc.subcore_barrier()
        pltpu.sync_copy(
            acc_sh.at[pl.ds(s * ROWS_PER_TILE, ROWS_PER_TILE)],
            out_hbm.at[c, pl.ds(s * ROWS_PER_TILE, ROWS_PER_TILE)])

    return _prop


_prop128 = _make_prop(H)
_prop64 = _make_prop(C_PAD)


_BR = 512
_GRID = N_PAD // _BR


def _pre1_body(x_ref, w_ref, degt_ref, u_ref, dinv_ref):
    deg = jnp.sum(degt_ref[...], axis=1, keepdims=True) + 1.0
    dinv = lax.rsqrt(deg)
    z = jnp.dot(x_ref[...], w_ref[...], preferred_element_type=jnp.float32)
    u_ref[...] = dinv * z
    dinv_ref[...] = dinv


def _tc_pre1(x_pad, W1, degT):
    return pl.pallas_call(
        _pre1_body,
        grid=(_GRID,),
        in_specs=[
            pl.BlockSpec((_BR, D), lambda i: (i, 0)),
            pl.BlockSpec((D, H), lambda i: (0, 0)),
            pl.BlockSpec((_BR, NW), lambda i: (i, 0)),
        ],
        out_specs=[
            pl.BlockSpec((_BR, H), lambda i: (i, 0)),
            pl.BlockSpec((_BR, 1), lambda i: (i, 0)),
        ],
        out_shape=[
            jax.ShapeDtypeStruct((N_PAD, H), jnp.float32),
            jax.ShapeDtypeStruct((N_PAD, 1), jnp.float32),
        ],
    )(x_pad, W1, degT)


def _mid_body(s0_ref, s1_ref, u_ref, dinv_ref, b_ref, w_ref, out_ref):
    dinv = dinv_ref[...]
    z = dinv * (s0_ref[...] + s1_ref[...] + u_ref[...]) + b_ref[...]
    h = jnp.maximum(z, 0.0)
    out_ref[...] = dinv * jnp.dot(h, w_ref[...],
                                  preferred_element_type=jnp.float32)


def _tc_mid(s0, s1, u, dinv, b, w, w_out):
    win = u.shape[1]
    return pl.pallas_call(
        _mid_body,
        grid=(_GRID,),
        in_specs=[
            pl.BlockSpec((_BR, win), lambda i: (i, 0)),
            pl.BlockSpec((_BR, win), lambda i: (i, 0)),
            pl.BlockSpec((_BR, win), lambda i: (i, 0)),
            pl.BlockSpec((_BR, 1), lambda i: (i, 0)),
            pl.BlockSpec((win,), lambda i: (0,)),
            pl.BlockSpec((win, w_out), lambda i: (0, 0)),
        ],
        out_specs=pl.BlockSpec((_BR, w_out), lambda i: (i, 0)),
        out_shape=jax.ShapeDtypeStruct((N_PAD, w_out), jnp.float32),
    )(s0, s1, u, dinv, b, w)


def _post_body(s0_ref, s1_ref, u_ref, dinv_ref, b_ref, out_ref):
    z = dinv_ref[...] * (s0_ref[...] + s1_ref[...] + u_ref[...]) + b_ref[...]
    m = jnp.max(z, axis=1, keepdims=True)
    e = jnp.exp(z - m)
    ssum = jnp.sum(e, axis=1, keepdims=True)
    out_ref[...] = z - m - jnp.log(ssum)


def _tc_post(s0, s1, u, dinv, b3p):
    return pl.pallas_call(
        _post_body,
        grid=(_GRID,),
        in_specs=[
            pl.BlockSpec((_BR, C_PAD), lambda i: (i, 0)),
            pl.BlockSpec((_BR, C_PAD), lambda i: (i, 0)),
            pl.BlockSpec((_BR, C_PAD), lambda i: (i, 0)),
            pl.BlockSpec((_BR, 1), lambda i: (i, 0)),
            pl.BlockSpec((C_PAD,), lambda i: (0,)),
        ],
        out_specs=pl.BlockSpec((_BR, C_PAD), lambda i: (i, 0)),
        out_shape=jax.ShapeDtypeStruct((N_PAD, C_PAD), jnp.float32),
    )(s0, s1, u, dinv, b3p)


def kernel(x, edge_index, W1, b1, W2, b2, W3, b3):
    pad_e = jnp.full((NW, EPW_PAD - EPW), N_PAD - 1, dtype=jnp.int32)
    src = jnp.concatenate(
        [edge_index[0].reshape(NW, EPW), pad_e], axis=1)
    dst = jnp.concatenate(
        [edge_index[1].reshape(NW, EPW), pad_e], axis=1)
    dst_flat = dst
    src = src.reshape(NW, CHUNKS, CHUNK)
    dst = dst.reshape(NW, CHUNKS, CHUNK)
    x_pad = jnp.pad(x, ((0, N_PAD - N), (0, 0)))
    W3p = jnp.pad(W3, ((0, 0), (0, C_PAD - C)))
    b3p = jnp.pad(b3, (0, C_PAD - C), constant_values=-1e30)

    degp = _deg_kernel(dst_flat)
    degT = degp.T

    u1, dinv = _tc_pre1(x_pad, W1, degT)
    s1 = _prop128(u1, src, dst)
    u2 = _tc_mid(s1[0], s1[1], u1, dinv, b1, W2, H)
    s2 = _prop128(u2, src, dst)
    u3 = _tc_mid(s2[0], s2[1], u2, dinv, b2, W3p, C_PAD)
    s3 = _prop64(u3, src, dst)
    out = _tc_post(s3[0], s3[1], u3, dinv, b3p)
    return out[:N, :C]
